# baseline (speedup 1.0000x reference)
import functools

import jax
import jax.numpy as jnp
from jax import lax
from jax.experimental import pallas as pl
from jax.experimental.pallas import tpu as pltpu
from jax.experimental.pallas import tpu_sc as plsc

NC = 2
NS = 16
NW = NC * NS
CHUNK = 40
NBUF = 4
PKSEG = 104
DCHUNK = 80


def _mesh():
  return plsc.VectorSubcoreMesh(
      core_axis_name="c", subcore_axis_name="s", num_cores=NC,
      num_subcores=NS)


def _zero_vmem_2d(ref, rows, width):
  zeros = jnp.zeros((16,), jnp.float32)

  @pl.loop(0, rows)
  def _(r):
    for k in range(width // 16):
      ref[r, pl.ds(k * 16, 16)] = zeros


def _deg_call(dst3d, nbins):
  nch = dst3d.shape[1]
  bpt = nbins // NS

  @functools.partial(
      pl.kernel,
      out_type=jax.ShapeDtypeStruct((NC * nbins,), jnp.float32),
      mesh=_mesh(),
      scratch_types=[
          pltpu.VMEM_SHARED((nbins,), jnp.float32),
          pltpu.VMEM((nch, DCHUNK), jnp.int32),
          pltpu.VMEM((DCHUNK,), jnp.float32),
          pltpu.VMEM((bpt,), jnp.float32),
          pltpu.SemaphoreType.DMA,
      ],
  )
  def k(dst_hbm, out_hbm, acc, didx, ones_v, zb, sem):
    c = lax.axis_index("c")
    s = lax.axis_index("s")
    wid = s * NC + c
    for i in range(DCHUNK // 16):
      ones_v[pl.ds(i * 16, 16)] = jnp.ones((16,), jnp.float32)

    @pl.loop(0, bpt // 16)
    def _(i):
      zb[pl.ds(i * 16, 16)] = jnp.zeros((16,), jnp.float32)

    pltpu.sync_copy(zb, acc.at[pl.ds(s * bpt, bpt)])
    pltpu.sync_copy(dst_hbm.at[wid], didx)
    plsc.subcore_barrier()

    bsz = 25
    assert nch % bsz == 0
    for base in range(0, nch, bsz):

      @pl.loop(base, base + bsz)
      def _(j):
        pltpu.async_copy(ones_v, acc.at[didx.at[j]], sem, add=True)

      @pl.loop(0, bsz)
      def _(j):
        pltpu.make_async_copy(out_hbm.at[pl.ds(0, DCHUNK)], ones_v,
                              sem).wait()

    plsc.subcore_barrier()
    pltpu.sync_copy(acc.at[pl.ds(s * bpt, bpt)],
                    out_hbm.at[pl.ds(c * nbins + s * bpt, bpt)])

  return k(dst3d)


def _scatter_call(h, pk3d, npad):
  n, d = h.shape
  nch = pk3d.shape[1]
  rpt = npad // NS
  segs = []
  off = 0
  while off < nch:
    seg = min(PKSEG, nch - off)
    assert seg >= 8
    segs.append((off, seg))
    off += seg

  @functools.partial(
      pl.kernel,
      out_type=jax.ShapeDtypeStruct((NC, npad, d), jnp.float32),
      mesh=_mesh(),
      scratch_types=[
          pltpu.VMEM_SHARED((npad, d), jnp.float32),
          pltpu.VMEM((PKSEG, CHUNK), jnp.int32),
          [pltpu.VMEM((CHUNK,), jnp.int32) for _ in range(NBUF)],
          [pltpu.VMEM((CHUNK,), jnp.int32) for _ in range(NBUF)],
          [pltpu.VMEM((CHUNK, d), jnp.float32) for _ in range(NBUF)],
          [pltpu.SemaphoreType.DMA for _ in range(NBUF)],
          [pltpu.SemaphoreType.DMA for _ in range(NBUF)],
      ],
  )
  def k(h_hbm, pk_hbm, out_hbm, acc, pk, sidx, didx, rows, semg, sems):
    c = lax.axis_index("c")
    s = lax.axis_index("s")
    wid = s * NC + c
    _zero_vmem_2d(rows[0], CHUNK, d)
    for t in range(rpt // CHUNK):
      pltpu.sync_copy(rows[0], acc.at[pl.ds(s * rpt + t * CHUNK, CHUNK)])
    plsc.subcore_barrier()

    def unpack(j, kk):
      for t in (0, 16, 24):
        p = pk[j, pl.ds(t, 16)]
        sidx[kk][pl.ds(t, 16)] = lax.bitwise_and(p, 0xFFFF)
        didx[kk][pl.ds(t, 16)] = lax.shift_right_logical(p, 16)

    def fire_g(kk):
      pltpu.async_copy(h_hbm.at[sidx[kk]], rows[kk], semg[kk])

    def wait_g(kk):
      pltpu.make_async_copy(h_hbm.at[pl.ds(0, CHUNK)], rows[kk],
                            semg[kk]).wait()

    def fire_s(kk):
      pltpu.async_copy(rows[kk], acc.at[didx[kk]], sems[kk], add=True)

    def drain_s(kk):
      pltpu.make_async_copy(h_hbm.at[pl.ds(0, CHUNK)], rows[kk],
                            sems[kk]).wait()

    def body(j, kk, fire_next, drain_prev):
      k2 = (kk + 2) % NBUF
      wait_g(kk)
      fire_s(kk)
      if drain_prev:
        drain_s(k2)
      if fire_next:
        unpack(j + 2, k2)
        fire_g(k2)

    for off, seg in segs:
      pltpu.sync_copy(pk_hbm.at[wid, pl.ds(off, seg)], pk.at[pl.ds(0, seg)])
      for j in range(2):
        unpack(j, j)
        fire_g(j)
      body(0, 0, True, False)
      body(1, 1, True, False)
      lp = ((seg - 4) // NBUF) * NBUF
      if lp > 0:

        @pl.loop(0, lp // NBUF)
        def _(m):
          j = NBUF * m + 2
          for t in range(NBUF):
            body(j + t, (2 + t) % NBUF, True, True)

      for j in range(2 + lp, seg - 2):
        body(j, j % NBUF, True, True)
      body(seg - 2, (seg - 2) % NBUF, False, False)
      body(seg - 1, (seg - 1) % NBUF, False, False)
      for t in range(NBUF):
        drain_s(t)
    plsc.subcore_barrier()
    pltpu.sync_copy(acc.at[pl.ds(s * rpt, rpt)],
                    out_hbm.at[c, pl.ds(s * rpt, rpt)])

  return k(h, pk3d)


def _pre_call(x, W1, dis_col):
  n = x.shape[0]
  h = W1.shape[1]

  def body(x_ref, w_ref, dis_ref, hw_ref, hpre_ref):
    hw = jnp.dot(x_ref[...], w_ref[...], preferred_element_type=jnp.float32)
    hw_ref[...] = hw
    hpre_ref[...] = hw * dis_ref[...]

  sh = jax.ShapeDtypeStruct((n, h), jnp.float32)
  return pl.pallas_call(body, out_shape=(sh, sh))(x, W1, dis_col)


def _mid_call(sp, hw, dis_col, b, gamma, beta, W2):
  n, h = hw.shape

  def body(sp_ref, hw_ref, dis_ref, b_ref, g_ref, be_ref, w2_ref,
           h2w_ref, h2pre_ref):
    dis = dis_ref[...]
    t = (dis * (sp_ref[0, :n] + sp_ref[1, :n]) + (dis * dis) * hw_ref[...]
         + b_ref[...])
    mean = jnp.mean(t, axis=0, keepdims=True)
    ctr = t - mean
    var = jnp.mean(ctr * ctr, axis=0, keepdims=True)
    hn = ctr * lax.rsqrt(var + 1e-5) * g_ref[...] + be_ref[...]
    hn = jnp.maximum(hn, 0.0)
    h2 = jnp.dot(hn, w2_ref[...], preferred_element_type=jnp.float32)
    h2w_ref[...] = h2
    h2pre_ref[...] = h2 * dis

  sh = jax.ShapeDtypeStruct((n, W2.shape[1]), jnp.float32)
  return pl.pallas_call(body, out_shape=(sh, sh))(
      sp, hw, dis_col, b, gamma, beta, W2)


def _post_call(sp, hw, dis_col, b, gamma, beta, W_mu, b_mu, W_lv, b_lv):
  n, h = hw.shape
  l = W_mu.shape[1]

  def body(sp_ref, hw_ref, dis_ref, b_ref, g_ref, be_ref, wmu_ref, bmu_ref,
           wlv_ref, blv_ref, mu_ref, lv_ref):
    dis = dis_ref[...]
    t = (dis * (sp_ref[0, :n] + sp_ref[1, :n]) + (dis * dis) * hw_ref[...]
         + b_ref[...])
    mean = jnp.mean(t, axis=0, keepdims=True)
    ctr = t - mean
    var = jnp.mean(ctr * ctr, axis=0, keepdims=True)
    hn = ctr * lax.rsqrt(var + 1e-5) * g_ref[...] + be_ref[...]
    hn = jnp.maximum(hn, 0.0)
    mu_ref[...] = (jnp.dot(hn, wmu_ref[...], preferred_element_type=jnp.float32)
                   + bmu_ref[...])
    lv_ref[...] = (jnp.dot(hn, wlv_ref[...], preferred_element_type=jnp.float32)
                   + blv_ref[...])

  sh = jax.ShapeDtypeStruct((n, l), jnp.float32)
  return pl.pallas_call(body, out_shape=(sh, sh))(
      sp, hw, dis_col, b, gamma, beta, W_mu, b_mu, W_lv, b_lv)


def kernel(x, edge_index, W1, b1, gamma1, beta1, W2, b2, gamma2, beta2,
           W_mu, b_mu, W_lv, b_lv):
  n = x.shape[0]
  e = edge_index.shape[1]
  assert e % (NW * CHUNK) == 0 and e % (NW * DCHUNK) == 0
  assert n % NS == 0 and n <= 65536
  npad = ((n + 2047) // 2048) * 2048
  nbins = npad

  dst3d = edge_index[1].reshape(NW, -1, DCHUNK)
  pk3d = (edge_index[0]
          | (edge_index[1] << jnp.int32(16))).reshape(NW, -1, CHUNK)

  deg_p = _deg_call(dst3d, nbins).reshape(NC, nbins)
  dis_col = lax.rsqrt(deg_p[0] + deg_p[1] + 1.0)[:n].reshape(n, 1)

  h1w, h1pre = _pre_call(x, W1, dis_col)
  s1p = _scatter_call(h1pre, pk3d, npad)
  h2w, h2pre = _mid_call(s1p, h1w, dis_col, b1, gamma1, beta1, W2)
  s2p = _scatter_call(h2pre, pk3d, npad)
  mu, logvar = _post_call(s2p, h2w, dis_col, b2, gamma2, beta2,
                          W_mu, b_mu, W_lv, b_lv)
  return (mu, logvar)


# speedup vs baseline: 1.0520x; 1.0520x over previous
import functools

import jax
import jax.numpy as jnp
from jax import lax
from jax.experimental import pallas as pl
from jax.experimental.pallas import tpu as pltpu
from jax.experimental.pallas import tpu_sc as plsc

NC = 2
NS = 16
NW = NC * NS
CHUNK = 40
NBUF = 3
DCHUNK = 80


def _mesh():
  return plsc.VectorSubcoreMesh(
      core_axis_name="c", subcore_axis_name="s", num_cores=NC,
      num_subcores=NS)


def _zero_vmem_2d(ref, rows, width):
  zeros = jnp.zeros((16,), jnp.float32)

  @pl.loop(0, rows)
  def _(r):
    for k in range(width // 16):
      ref[r, pl.ds(k * 16, 16)] = zeros


def _deg_call(dst3d, nbins):
  nch = dst3d.shape[1]
  bpt = nbins // NS

  @functools.partial(
      pl.kernel,
      out_type=jax.ShapeDtypeStruct((NC * nbins,), jnp.float32),
      mesh=_mesh(),
      scratch_types=[
          pltpu.VMEM_SHARED((nbins,), jnp.float32),
          pltpu.VMEM((nch, DCHUNK), jnp.int32),
          pltpu.VMEM((DCHUNK,), jnp.float32),
          pltpu.VMEM((bpt,), jnp.float32),
          pltpu.SemaphoreType.DMA,
      ],
  )
  def k(dst_hbm, out_hbm, acc, didx, ones_v, zb, sem):
    c = lax.axis_index("c")
    s = lax.axis_index("s")
    wid = s * NC + c
    for i in range(DCHUNK // 16):
      ones_v[pl.ds(i * 16, 16)] = jnp.ones((16,), jnp.float32)

    @pl.loop(0, bpt // 16)
    def _(i):
      zb[pl.ds(i * 16, 16)] = jnp.zeros((16,), jnp.float32)

    pltpu.sync_copy(zb, acc.at[pl.ds(s * bpt, bpt)])
    pltpu.sync_copy(dst_hbm.at[wid], didx)
    plsc.subcore_barrier()

    bsz = 25
    assert nch % bsz == 0
    for base in range(0, nch, bsz):

      @pl.loop(base, base + bsz)
      def _(j):
        pltpu.async_copy(ones_v, acc.at[didx.at[j]], sem, add=True)

      @pl.loop(0, bsz)
      def _(j):
        pltpu.make_async_copy(out_hbm.at[pl.ds(0, DCHUNK)], ones_v,
                              sem).wait()

    plsc.subcore_barrier()
    pltpu.sync_copy(acc.at[pl.ds(s * bpt, bpt)],
                    out_hbm.at[pl.ds(c * nbins + s * bpt, bpt)])

  return k(dst3d)


def _scatter_call(h, pk3d, npad):
  n, d = h.shape
  nch = pk3d.shape[1]
  assert (nch - 7) % NBUF == 0
  rpt = npad // NS

  @functools.partial(
      pl.kernel,
      out_type=jax.ShapeDtypeStruct((NC, npad, d), jnp.float32),
      mesh=_mesh(),
      scratch_types=[
          pltpu.VMEM_SHARED((npad, d), jnp.float32),
          pltpu.VMEM((nch, CHUNK), jnp.int32),
          [pltpu.VMEM((CHUNK,), jnp.int32) for _ in range(NBUF)],
          [pltpu.VMEM((CHUNK,), jnp.int32) for _ in range(NBUF)],
          [pltpu.VMEM((CHUNK, d), jnp.float32) for _ in range(NBUF)],
          [pltpu.SemaphoreType.DMA for _ in range(NBUF)],
          [pltpu.SemaphoreType.DMA for _ in range(NBUF)],
      ],
  )
  def k(h_hbm, pk_hbm, out_hbm, acc, pk, sidx, didx, rows, semg, sems):
    c = lax.axis_index("c")
    s = lax.axis_index("s")
    wid = s * NC + c
    _zero_vmem_2d(rows[0], CHUNK, d)
    for t in range(rpt // CHUNK):
      pltpu.sync_copy(rows[0], acc.at[pl.ds(s * rpt + t * CHUNK, CHUNK)])
    pltpu.sync_copy(pk_hbm.at[wid], pk)
    plsc.subcore_barrier()

    def unpack(j, kk):
      for t in (0, 16, 24):
        p = pk[j, pl.ds(t, 16)]
        sidx[kk][pl.ds(t, 16)] = lax.bitwise_and(p, 0xFFFF)
        didx[kk][pl.ds(t, 16)] = lax.shift_right_logical(p, 16)

    def fire_g(kk):
      pltpu.async_copy(h_hbm.at[sidx[kk]], rows[kk], semg[kk])

    def wait_g(kk):
      pltpu.make_async_copy(h_hbm.at[pl.ds(0, CHUNK)], rows[kk],
                            semg[kk]).wait()

    def fire_s(kk):
      pltpu.async_copy(rows[kk], acc.at[didx[kk]], sems[kk], add=True)

    def drain_s(kk):
      pltpu.make_async_copy(h_hbm.at[pl.ds(0, CHUNK)], rows[kk],
                            sems[kk]).wait()

    def body(j, kk, fire_next, drain_prev):
      k2 = (kk + 2) % NBUF
      wait_g(kk)
      fire_s(kk)
      if drain_prev:
        drain_s(k2)
      if fire_next:
        unpack(j + 2, k2)
        fire_g(k2)

    for j in range(2):
      unpack(j, j)
      fire_g(j)
    body(0, 0, True, False)

    @pl.loop(0, (nch - 7) // NBUF)
    def _(m):
      j = NBUF * m + 1
      for t in range(NBUF):
        body(j + t, (1 + t) % NBUF, True, True)

    for j in range(nch - 6, nch - 2):
      body(j, j % NBUF, True, True)
    body(nch - 2, (nch - 2) % NBUF, False, False)
    body(nch - 1, (nch - 1) % NBUF, False, False)
    for t in range(NBUF):
      drain_s(t)
    plsc.subcore_barrier()
    pltpu.sync_copy(acc.at[pl.ds(s * rpt, rpt)],
                    out_hbm.at[c, pl.ds(s * rpt, rpt)])

  return k(h, pk3d)


def _pre_call(x, W1, dis_col):
  n = x.shape[0]
  h = W1.shape[1]

  def body(x_ref, w_ref, dis_ref, hw_ref, hpre_ref):
    hw = jnp.dot(x_ref[...], w_ref[...], preferred_element_type=jnp.float32)
    hw_ref[...] = hw
    hpre_ref[...] = hw * dis_ref[...]

  sh = jax.ShapeDtypeStruct((n, h), jnp.float32)
  return pl.pallas_call(body, out_shape=(sh, sh))(x, W1, dis_col)


def _mid_call(sp, hw, dis_col, b, gamma, beta, W2):
  n, h = hw.shape

  def body(sp_ref, hw_ref, dis_ref, b_ref, g_ref, be_ref, w2_ref,
           h2w_ref, h2pre_ref):
    dis = dis_ref[...]
    t = (dis * (sp_ref[0, :n] + sp_ref[1, :n]) + (dis * dis) * hw_ref[...]
         + b_ref[...])
    mean = jnp.mean(t, axis=0, keepdims=True)
    ctr = t - mean
    var = jnp.mean(ctr * ctr, axis=0, keepdims=True)
    hn = ctr * lax.rsqrt(var + 1e-5) * g_ref[...] + be_ref[...]
    hn = jnp.maximum(hn, 0.0)
    h2 = jnp.dot(hn, w2_ref[...], preferred_element_type=jnp.float32)
    h2w_ref[...] = h2
    h2pre_ref[...] = h2 * dis

  sh = jax.ShapeDtypeStruct((n, W2.shape[1]), jnp.float32)
  return pl.pallas_call(body, out_shape=(sh, sh))(
      sp, hw, dis_col, b, gamma, beta, W2)


def _post_call(sp, hw, dis_col, b, gamma, beta, W_mu, b_mu, W_lv, b_lv):
  n, h = hw.shape
  l = W_mu.shape[1]

  def body(sp_ref, hw_ref, dis_ref, b_ref, g_ref, be_ref, wmu_ref, bmu_ref,
           wlv_ref, blv_ref, mu_ref, lv_ref):
    dis = dis_ref[...]
    t = (dis * (sp_ref[0, :n] + sp_ref[1, :n]) + (dis * dis) * hw_ref[...]
         + b_ref[...])
    mean = jnp.mean(t, axis=0, keepdims=True)
    ctr = t - mean
    var = jnp.mean(ctr * ctr, axis=0, keepdims=True)
    hn = ctr * lax.rsqrt(var + 1e-5) * g_ref[...] + be_ref[...]
    hn = jnp.maximum(hn, 0.0)
    mu_ref[...] = (jnp.dot(hn, wmu_ref[...], preferred_element_type=jnp.float32)
                   + bmu_ref[...])
    lv_ref[...] = (jnp.dot(hn, wlv_ref[...], preferred_element_type=jnp.float32)
                   + blv_ref[...])

  sh = jax.ShapeDtypeStruct((n, l), jnp.float32)
  return pl.pallas_call(body, out_shape=(sh, sh))(
      sp, hw, dis_col, b, gamma, beta, W_mu, b_mu, W_lv, b_lv)


def kernel(x, edge_index, W1, b1, gamma1, beta1, W2, b2, gamma2, beta2,
           W_mu, b_mu, W_lv, b_lv):
  n = x.shape[0]
  e = edge_index.shape[1]
  assert e % (NW * CHUNK) == 0 and e % (NW * DCHUNK) == 0
  assert n % NS == 0 and n <= 65536
  npad = ((n + 2047) // 2048) * 2048
  nbins = npad

  dst3d = edge_index[1].reshape(NW, -1, DCHUNK)
  pk3d = (edge_index[0]
          | (edge_index[1] << jnp.int32(16))).reshape(NW, -1, CHUNK)

  deg_p = _deg_call(dst3d, nbins).reshape(NC, nbins)
  dis_col = lax.rsqrt(deg_p[0] + deg_p[1] + 1.0)[:n].reshape(n, 1)

  h1w, h1pre = _pre_call(x, W1, dis_col)
  s1p = _scatter_call(h1pre, pk3d, npad)
  h2w, h2pre = _mid_call(s1p, h1w, dis_col, b1, gamma1, beta1, W2)
  s2p = _scatter_call(h2pre, pk3d, npad)
  mu, logvar = _post_call(s2p, h2w, dis_col, b2, gamma2, beta2,
                          W_mu, b_mu, W_lv, b_lv)
  return (mu, logvar)
